# baseline (speedup 1.0000x reference)
import functools

import jax
import jax.numpy as jnp
import numpy as np
from jax import lax
from jax.experimental import pallas as pl
from jax.experimental.pallas import tpu as pltpu
from jax.experimental.pallas import tpu_sc as plsc

B, N, C = 1, 2048, 768
H, D = 12, 64
TOKEEP = 50
WIN = 4
SCALE_P = 3.0
ALPHA_P = 2.0

NW_CHUNKS = N // WIN
NSEL = H * NW_CHUNKS * TOKEEP
PB_ROWS = 2 * N
CB = 16
BM = CB * WIN
BZ = CB * TOKEEP
NBLK = NW_CHUNKS // CB

SC_NW = 32
PER_W = NSEL // SC_NW
KV_G = 120
KV_NG = PER_W // KV_G

_CONSTS = {}


def _np_exponential_key1234(shape):
    size = int(np.prod(shape))
    k1, k2 = np.uint32(0), np.uint32(1234)
    c64 = np.arange(size, dtype=np.uint64)
    x0 = (c64 >> np.uint64(32)).astype(np.uint32)
    x1 = (c64 & np.uint64(0xFFFFFFFF)).astype(np.uint32)

    def rotl(x, d):
        d = np.uint32(d)
        return (x << d) | (x >> np.uint32(32 - d))

    ks0, ks1 = k1, k2
    ks2 = np.uint32(k1 ^ k2 ^ np.uint32(0x1BD11BDA))
    x0 = x0 + ks0
    x1 = x1 + ks1

    def rounds(x0, x1, rots, ka, kb, i):
        for r in rots:
            x0 = x0 + x1
            x1 = rotl(x1, r) ^ x0
        return x0 + ka, x1 + np.uint32(kb + np.uint32(i))

    rot1, rot2 = (13, 15, 26, 6), (17, 29, 16, 24)
    x0, x1 = rounds(x0, x1, rot1, ks1, ks2, 1)
    x0, x1 = rounds(x0, x1, rot2, ks2, ks0, 2)
    x0, x1 = rounds(x0, x1, rot1, ks0, ks1, 3)
    x0, x1 = rounds(x0, x1, rot2, ks1, ks2, 4)
    x0, x1 = rounds(x0, x1, rot1, ks2, ks0, 5)
    bits = (x0 ^ x1).reshape(shape)
    float_bits = (bits >> np.uint32(9)) | np.uint32(0x3F800000)
    u = float_bits.view(np.float32) - np.float32(1.0)
    return -np.log1p(-u)


def _get_consts():
    if _CONSTS:
        return _CONSTS
    grid = np.repeat(
        np.abs(np.arange(NW_CHUNKS)[None, :] - np.arange(NW_CHUNKS)[:, None]),
        WIN, axis=1).astype(np.float32)
    e = _np_exponential_key1234((B, H, NW_CHUNKS, N)) / np.float32(ALPHA_P)
    pareto = (np.float32(SCALE_P) * np.exp(e.astype(np.float32))).astype(np.float32)
    cg = grid[None, None, :, :] - pareto
    idx = np.sort(np.argsort(cg, axis=-1, kind="stable")[..., :TOKEEP], axis=-1)
    idx = idx[0].astype(np.int64)

    hh = np.arange(H)[:, None, None]
    cc = np.arange(NW_CHUNKS)[None, :, None]
    gidx = (idx * H + hh).reshape(-1).astype(np.int32)
    pbase = (hh * PB_ROWS + (WIN * cc - idx + (N - 1))).reshape(-1).astype(np.int32)
    mask = np.full((BM, BZ), -1e30, np.float32)
    for c in range(CB):
        mask[c * WIN:(c + 1) * WIN, c * TOKEEP:(c + 1) * TOKEEP] = 0.0
    i4 = np.zeros((BM, WIN), np.float32)
    i4[np.arange(BM), np.arange(BM) % WIN] = 1.0
    _CONSTS.update(gidx=gidx, pbase=pbase, mask=mask, i4=i4)
    return _CONSTS


def _qkv_body(x_ref, w_ref, b_ref, q_ref, kv_ref):
    o = lax.dot_general(x_ref[...], w_ref[...], (((1,), (1,)), ((), ())),
                        preferred_element_type=jnp.float32)
    o = o + b_ref[...]
    for h in range(H):
        q_ref[h] = o[:, h * D:(h + 1) * D]
    kv_ref[...] = o[:, C:]


def _qkv_proj(x2d, w_r, b_r):
    m_blk = 256
    return pl.pallas_call(
        _qkv_body,
        grid=(N // m_blk,),
        in_specs=[
            pl.BlockSpec((m_blk, C), lambda i: (i, 0)),
            pl.BlockSpec((3 * C, C), lambda i: (0, 0)),
            pl.BlockSpec((1, 3 * C), lambda i: (0, 0)),
        ],
        out_specs=[
            pl.BlockSpec((H, m_blk, D), lambda i: (0, i, 0)),
            pl.BlockSpec((m_blk, 2 * C), lambda i: (i, 0)),
        ],
        out_shape=[
            jax.ShapeDtypeStruct((H, N, D), jnp.float32),
            jax.ShapeDtypeStruct((N, 2 * C), jnp.float32),
        ],
    )(x2d, w_r, b_r)


def _mlp_body(w0_ref, b0_ref, w1_ref, b1_ref, w2_ref, b2_ref, o_ref):
    i = pl.program_id(0)
    r_blk = o_ref.shape[0]
    rows = lax.broadcasted_iota(jnp.int32, (r_blk, C), 0)
    pos = rows.astype(jnp.float32) + (i * r_blk - (N - 1)).astype(jnp.float32)
    h1 = jnp.maximum(pos * w0_ref[...] + b0_ref[...], 0.0)
    h2 = lax.dot_general(h1, w1_ref[...], (((1,), (1,)), ((), ())),
                         preferred_element_type=jnp.float32)
    h2 = jnp.maximum(h2 + b1_ref[...], 0.0)
    o = lax.dot_general(h2, w2_ref[...], (((1,), (1,)), ((), ())),
                        preferred_element_type=jnp.float32)
    o_ref[...] = o + b2_ref[...]


def _pos_mlp(w0r, b0r, w1t, b1r, w2t, b2r):
    r_blk = 1024
    return pl.pallas_call(
        _mlp_body,
        grid=(PB_ROWS // r_blk,),
        in_specs=[
            pl.BlockSpec((1, C), lambda i: (0, 0)),
            pl.BlockSpec((1, C), lambda i: (0, 0)),
            pl.BlockSpec((C, C), lambda i: (0, 0)),
            pl.BlockSpec((1, C), lambda i: (0, 0)),
            pl.BlockSpec((H, C), lambda i: (0, 0)),
            pl.BlockSpec((1, H), lambda i: (0, 0)),
        ],
        out_specs=pl.BlockSpec((r_blk, H), lambda i: (i, 0)),
        out_shape=jax.ShapeDtypeStruct((PB_ROWS, H), jnp.float32),
    )(w0r, b0r, w1t, b1r, w2t, b2r)


def _attn_body(q_ref, kv_ref, p_ref, m_ref, i4_ref, o_ref):
    q = q_ref[...].reshape(BM, D)
    kv = kv_ref[...]
    k = kv[:, :D]
    v = kv[:, D:]
    s = lax.dot_general(q, k, (((1,), (1,)), ((), ())),
                        preferred_element_type=jnp.float32)
    s = s * (D ** -0.5)
    p = p_ref[...].reshape(WIN, BZ)
    s = s + jnp.dot(i4_ref[...], p, preferred_element_type=jnp.float32)
    s = s + m_ref[...]
    mx = jnp.max(s, axis=1, keepdims=True)
    e = jnp.exp(s - mx)
    sm = jnp.sum(e, axis=1, keepdims=True)
    attn = e / sm
    o = jnp.dot(attn, v, preferred_element_type=jnp.float32)
    o_ref[...] = o.reshape(1, BM, D)


def _attention(q3, kvsel, psel4, mask, i4):
    return pl.pallas_call(
        _attn_body,
        grid=(H, NBLK),
        in_specs=[
            pl.BlockSpec((1, BM, D), lambda h, b: (h, b, 0)),
            pl.BlockSpec((BZ, 2 * D), lambda h, b: (h * NBLK + b, 0)),
            pl.BlockSpec((1, WIN, BZ), lambda h, b: (h * NBLK + b, 0, 0)),
            pl.BlockSpec((BM, BZ), lambda h, b: (0, 0)),
            pl.BlockSpec((BM, WIN), lambda h, b: (0, 0)),
        ],
        out_specs=pl.BlockSpec((1, BM, D), lambda h, b: (h, b, 0)),
        out_shape=jax.ShapeDtypeStruct((H, N, D), jnp.float32),
    )(q3, kvsel, psel4, mask, i4)


def _proj_body(a_ref, w_ref, b_ref, o_ref):
    x = jnp.concatenate([a_ref[h] for h in range(H)], axis=1)
    o = lax.dot_general(x, w_ref[...], (((1,), (1,)), ((), ())),
                        preferred_element_type=jnp.float32)
    o_ref[...] = o + b_ref[...]


def _out_proj(a3, w, b_r):
    m_blk = 512
    return pl.pallas_call(
        _proj_body,
        grid=(N // m_blk,),
        in_specs=[
            pl.BlockSpec((H, m_blk, D), lambda i: (0, i, 0)),
            pl.BlockSpec((C, C), lambda i: (0, 0)),
            pl.BlockSpec((1, C), lambda i: (0, 0)),
        ],
        out_specs=pl.BlockSpec((m_blk, C), lambda i: (i, 0)),
        out_shape=jax.ShapeDtypeStruct((N, C), jnp.float32),
    )(a3, w, b_r)


def _sc_mesh():
    return plsc.VectorSubcoreMesh(core_axis_name="c", subcore_axis_name="s")


def _sc_wid():
    return lax.axis_index("s") * 2 + lax.axis_index("c")


def _kv_gather(kvtab, gidx):
    nbuf = 4

    @functools.partial(
        pl.kernel,
        out_type=jax.ShapeDtypeStruct((NSEL, 2 * D), jnp.float32),
        mesh=_sc_mesh(),
        scratch_types=[
            pltpu.VMEM((PER_W,), jnp.int32),
        ] + [pltpu.VMEM((KV_G, 2 * D), jnp.float32) for _ in range(nbuf)]
          + [pltpu.SemaphoreType.DMA for _ in range(nbuf)]
          + [pltpu.SemaphoreType.DMA for _ in range(nbuf)],
    )
    def body(tab_hbm, idx_hbm, out_hbm, idx_v, *bufsem):
        bufs = bufsem[:nbuf]
        gsem = bufsem[nbuf:2 * nbuf]
        osem = bufsem[2 * nbuf:]
        base = _sc_wid() * PER_W
        pltpu.sync_copy(idx_hbm.at[pl.ds(base, PER_W)], idx_v)

        def gather(t, b):
            return pltpu.make_async_copy(
                tab_hbm.at[idx_v.at[pl.ds(t * KV_G, KV_G)]], bufs[b], gsem[b])

        def out_cp(t, b):
            return pltpu.make_async_copy(
                bufs[b], out_hbm.at[pl.ds(base + t * KV_G, KV_G)], osem[b])

        for b in range(nbuf):
            gather(b, b).start()

        def step(g, _):
            for b in range(nbuf):
                t = g * nbuf + b
                gather(t, b).wait()
                out_cp(t, b).start()
                nxt = t + nbuf

                @pl.when(nxt < KV_NG)
                def _(b=b, t=t, nxt=nxt):
                    out_cp(t, b).wait()
                    gather(nxt, b).start()
            return 0

        lax.fori_loop(0, KV_NG // nbuf, step, 0)
        for b in range(nbuf):
            out_cp(KV_NG - nbuf + b, b).wait()

    return body(kvtab, gidx)


def _pos_gather(pbh_t, pbase):
    blks = PER_W // BZ

    @functools.partial(
        pl.kernel,
        out_type=jax.ShapeDtypeStruct((H * NBLK, WIN, BZ), jnp.float32),
        mesh=_sc_mesh(),
        scratch_types=[
            pltpu.VMEM((H * PB_ROWS,), jnp.float32),
            pltpu.VMEM((PER_W,), jnp.int32),
            pltpu.VMEM((blks, WIN, BZ), jnp.float32),
        ],
        compiler_params=pltpu.CompilerParams(needs_layout_passes=False),
    )
    def body(tab_hbm, idx_hbm, out_hbm, tab_v, idx_v, out_v):
        wid = _sc_wid()
        pltpu.sync_copy(tab_hbm, tab_v)
        pltpu.sync_copy(idx_hbm.at[pl.ds(wid * PER_W, PER_W)], idx_v)

        for blk in range(blks):
            def step(t, _, blk=blk):
                iv = idx_v[pl.ds((blk * (BZ // 16) + t) * 16, 16)]
                for j in range(WIN):
                    out_v[blk, j, pl.ds(t * 16, 16)] = \
                        plsc.load_gather(tab_v, [iv + j])
                return 0

            lax.fori_loop(0, BZ // 16, step, 0)
        pltpu.sync_copy(out_v, out_hbm.at[pl.ds(wid * blks, blks)])

    return body(pbh_t, pbase)


def kernel(x, qkv_w, qkv_b, out_w, out_b, pb_w0, pb_b0, pb_w1, pb_b1, pb_w2, pb_b2):
    cst = _get_consts()
    gidx = jnp.asarray(cst["gidx"])
    pbase = jnp.asarray(cst["pbase"])
    mask = jnp.asarray(cst["mask"])
    i4 = jnp.asarray(cst["i4"])

    w4 = qkv_w.reshape(H, D, 3, C)
    qw = w4[:, :, 0, :].reshape(H * D, C)
    kvw = jnp.transpose(w4[:, :, 1:, :], (0, 2, 1, 3)).reshape(2 * H * D, C)
    w_r = jnp.concatenate([qw, kvw], axis=0)
    b4 = qkv_b.reshape(H, D, 3)
    qb = b4[:, :, 0].reshape(-1)
    kvb = jnp.transpose(b4[:, :, 1:], (0, 2, 1)).reshape(-1)
    b_r = jnp.concatenate([qb, kvb])[None, :]

    q3, kv2d = _qkv_proj(x[0], w_r, b_r)
    kvtab = kv2d.reshape(N * H, 2 * D)

    pbh = _pos_mlp(pb_w0.T, pb_b0[None, :], pb_w1, pb_b1[None, :],
                   pb_w2, pb_b2[None, :])
    pbh_t = pbh.T.reshape(-1)

    kvsel = _kv_gather(kvtab, gidx)
    psel4 = _pos_gather(pbh_t, pbase)

    a3 = _attention(q3, kvsel, psel4, mask, i4)
    y = _out_proj(a3, out_w, out_b[None, :])
    return y[None]


# speedup vs baseline: 1.3022x; 1.3022x over previous
import functools

import jax
import jax.numpy as jnp
import numpy as np
from jax import lax
from jax.experimental import pallas as pl
from jax.experimental.pallas import tpu as pltpu
from jax.experimental.pallas import tpu_sc as plsc

B, N, C = 1, 2048, 768
H, D = 12, 64
TOKEEP = 50
WIN = 4
SCALE_P = 3.0
ALPHA_P = 2.0

NW_CHUNKS = N // WIN
NSEL = H * NW_CHUNKS * TOKEEP
PB_ROWS = 2 * N
CB = 32
BM = CB * WIN
BZ = CB * TOKEEP
NBLK = NW_CHUNKS // CB

SC_NW = 32
PER_W = NSEL // SC_NW
KV_G = 120
KV_NG = PER_W // KV_G

_CONSTS = {}


def _np_exponential_key1234(shape):
    size = int(np.prod(shape))
    k1, k2 = np.uint32(0), np.uint32(1234)
    c64 = np.arange(size, dtype=np.uint64)
    x0 = (c64 >> np.uint64(32)).astype(np.uint32)
    x1 = (c64 & np.uint64(0xFFFFFFFF)).astype(np.uint32)

    def rotl(x, d):
        d = np.uint32(d)
        return (x << d) | (x >> np.uint32(32 - d))

    ks0, ks1 = k1, k2
    ks2 = np.uint32(k1 ^ k2 ^ np.uint32(0x1BD11BDA))
    x0 = x0 + ks0
    x1 = x1 + ks1

    def rounds(x0, x1, rots, ka, kb, i):
        for r in rots:
            x0 = x0 + x1
            x1 = rotl(x1, r) ^ x0
        return x0 + ka, x1 + np.uint32(kb + np.uint32(i))

    rot1, rot2 = (13, 15, 26, 6), (17, 29, 16, 24)
    x0, x1 = rounds(x0, x1, rot1, ks1, ks2, 1)
    x0, x1 = rounds(x0, x1, rot2, ks2, ks0, 2)
    x0, x1 = rounds(x0, x1, rot1, ks0, ks1, 3)
    x0, x1 = rounds(x0, x1, rot2, ks1, ks2, 4)
    x0, x1 = rounds(x0, x1, rot1, ks2, ks0, 5)
    bits = (x0 ^ x1).reshape(shape)
    float_bits = (bits >> np.uint32(9)) | np.uint32(0x3F800000)
    u = float_bits.view(np.float32) - np.float32(1.0)
    return -np.log1p(-u)


def _get_consts():
    if _CONSTS:
        return _CONSTS
    grid = np.repeat(
        np.abs(np.arange(NW_CHUNKS)[None, :] - np.arange(NW_CHUNKS)[:, None]),
        WIN, axis=1).astype(np.float32)
    e = _np_exponential_key1234((B, H, NW_CHUNKS, N)) / np.float32(ALPHA_P)
    pareto = (np.float32(SCALE_P) * np.exp(e.astype(np.float32))).astype(np.float32)
    cg = grid[None, None, :, :] - pareto
    idx = np.sort(np.argsort(cg, axis=-1, kind="stable")[..., :TOKEEP], axis=-1)
    idx = idx[0].astype(np.int64)

    hh = np.arange(H)[:, None, None]
    cc = np.arange(NW_CHUNKS)[None, :, None]
    gidx = (idx * H + hh).reshape(-1).astype(np.int32)
    pbase = (hh * PB_ROWS + (WIN * cc - idx + (N - 1))).reshape(-1).astype(np.int32)
    mask = np.full((BM, BZ), -1e30, np.float32)
    for c in range(CB):
        mask[c * WIN:(c + 1) * WIN, c * TOKEEP:(c + 1) * TOKEEP] = 0.0
    i4 = np.zeros((BM, WIN), np.float32)
    i4[np.arange(BM), np.arange(BM) % WIN] = 1.0
    _CONSTS.update(gidx=gidx, pbase=pbase, mask=mask, i4=i4)
    return _CONSTS


def _qkv_body(x_ref, w_ref, b_ref, q_ref, kv_ref):
    o = lax.dot_general(x_ref[...], w_ref[...], (((1,), (1,)), ((), ())),
                        preferred_element_type=jnp.float32)
    o = o + b_ref[...]
    for h in range(H):
        q_ref[h] = o[:, h * D:(h + 1) * D]
    kv_ref[...] = o[:, C:]


def _qkv_proj(x2d, w_r, b_r):
    m_blk = 256
    return pl.pallas_call(
        _qkv_body,
        grid=(N // m_blk,),
        in_specs=[
            pl.BlockSpec((m_blk, C), lambda i: (i, 0)),
            pl.BlockSpec((3 * C, C), lambda i: (0, 0)),
            pl.BlockSpec((1, 3 * C), lambda i: (0, 0)),
        ],
        out_specs=[
            pl.BlockSpec((H, m_blk, D), lambda i: (0, i, 0)),
            pl.BlockSpec((m_blk, 2 * C), lambda i: (i, 0)),
        ],
        out_shape=[
            jax.ShapeDtypeStruct((H, N, D), jnp.float32),
            jax.ShapeDtypeStruct((N, 2 * C), jnp.float32),
        ],
    )(x2d, w_r, b_r)


def _mlp_body(w0_ref, b0_ref, w1_ref, b1_ref, w2_ref, b2_ref, o_ref):
    i = pl.program_id(0)
    r_blk = o_ref.shape[0]
    rows = lax.broadcasted_iota(jnp.int32, (r_blk, C), 0)
    pos = rows.astype(jnp.float32) + (i * r_blk - (N - 1)).astype(jnp.float32)
    h1 = jnp.maximum(pos * w0_ref[...] + b0_ref[...], 0.0)
    h2 = lax.dot_general(h1, w1_ref[...], (((1,), (1,)), ((), ())),
                         preferred_element_type=jnp.float32)
    h2 = jnp.maximum(h2 + b1_ref[...], 0.0)
    o = lax.dot_general(h2, w2_ref[...], (((1,), (1,)), ((), ())),
                        preferred_element_type=jnp.float32)
    o_ref[...] = o + b2_ref[...]


def _pos_mlp(w0r, b0r, w1t, b1r, w2t, b2r):
    r_blk = 1024
    return pl.pallas_call(
        _mlp_body,
        grid=(PB_ROWS // r_blk,),
        in_specs=[
            pl.BlockSpec((1, C), lambda i: (0, 0)),
            pl.BlockSpec((1, C), lambda i: (0, 0)),
            pl.BlockSpec((C, C), lambda i: (0, 0)),
            pl.BlockSpec((1, C), lambda i: (0, 0)),
            pl.BlockSpec((H, C), lambda i: (0, 0)),
            pl.BlockSpec((1, H), lambda i: (0, 0)),
        ],
        out_specs=pl.BlockSpec((r_blk, H), lambda i: (i, 0)),
        out_shape=jax.ShapeDtypeStruct((PB_ROWS, H), jnp.float32),
    )(w0r, b0r, w1t, b1r, w2t, b2r)


def _attn_body(q_ref, kv_ref, p_ref, m_ref, i4_ref, o_ref):
    q = q_ref[...].reshape(BM, D)
    kv = kv_ref[...]
    k = kv[:, :D]
    v = kv[:, D:]
    s = lax.dot_general(q, k, (((1,), (1,)), ((), ())),
                        preferred_element_type=jnp.float32)
    s = s * (D ** -0.5)
    p = p_ref[...].reshape(WIN, BZ)
    s = s + jnp.dot(i4_ref[...], p, preferred_element_type=jnp.float32)
    s = s + m_ref[...]
    mx = jnp.max(s, axis=1, keepdims=True)
    e = jnp.exp(s - mx)
    sm = jnp.sum(e, axis=1, keepdims=True)
    attn = e / sm
    o = jnp.dot(attn, v, preferred_element_type=jnp.float32)
    o_ref[...] = o.reshape(1, BM, D)


def _attention(q3, kvsel, psel4, mask, i4, h0, nh):
    return pl.pallas_call(
        _attn_body,
        grid=(nh, NBLK),
        in_specs=[
            pl.BlockSpec((1, BM, D), lambda h, b: (h0 + h, b, 0)),
            pl.BlockSpec((BZ, 2 * D), lambda h, b: (h * NBLK + b, 0)),
            pl.BlockSpec((1, WIN, BZ),
                         lambda h, b: ((h0 + h) * NBLK + b, 0, 0)),
            pl.BlockSpec((BM, BZ), lambda h, b: (0, 0)),
            pl.BlockSpec((BM, WIN), lambda h, b: (0, 0)),
        ],
        out_specs=pl.BlockSpec((1, BM, D), lambda h, b: (h, b, 0)),
        out_shape=jax.ShapeDtypeStruct((nh, N, D), jnp.float32),
    )(q3, kvsel, psel4, mask, i4)


def _proj_body(a_ref, b_ref2, w_ref, b_ref, o_ref):
    nh = a_ref.shape[0]
    x = jnp.concatenate([a_ref[h] for h in range(nh)]
                        + [b_ref2[h] for h in range(H - nh)], axis=1)
    o = lax.dot_general(x, w_ref[...], (((1,), (1,)), ((), ())),
                        preferred_element_type=jnp.float32)
    o_ref[...] = o + b_ref[...]


def _out_proj(a3a, a3b, w, b_r):
    m_blk = 512
    nha = a3a.shape[0]
    nhb = a3b.shape[0]
    return pl.pallas_call(
        _proj_body,
        grid=(N // m_blk,),
        in_specs=[
            pl.BlockSpec((nha, m_blk, D), lambda i: (0, i, 0)),
            pl.BlockSpec((nhb, m_blk, D), lambda i: (0, i, 0)),
            pl.BlockSpec((C, C), lambda i: (0, 0)),
            pl.BlockSpec((1, C), lambda i: (0, 0)),
        ],
        out_specs=pl.BlockSpec((m_blk, C), lambda i: (i, 0)),
        out_shape=jax.ShapeDtypeStruct((N, C), jnp.float32),
    )(a3a, a3b, w, b_r)


def _sc_mesh():
    return plsc.VectorSubcoreMesh(core_axis_name="c", subcore_axis_name="s")


def _sc_wid():
    return lax.axis_index("s") * 2 + lax.axis_index("c")


def _kv_gather(kvtab, gidx, nsel):
    nbuf = 4
    per_w = nsel // SC_NW
    n_g = per_w // KV_G

    @functools.partial(
        pl.kernel,
        out_type=jax.ShapeDtypeStruct((nsel, 2 * D), jnp.float32),
        mesh=_sc_mesh(),
        scratch_types=[
            pltpu.VMEM((per_w,), jnp.int32),
        ] + [pltpu.VMEM((KV_G, 2 * D), jnp.float32) for _ in range(nbuf)]
          + [pltpu.SemaphoreType.DMA for _ in range(nbuf)]
          + [pltpu.SemaphoreType.DMA for _ in range(nbuf)],
    )
    def body(tab_hbm, idx_hbm, out_hbm, idx_v, *bufsem):
        bufs = bufsem[:nbuf]
        gsem = bufsem[nbuf:2 * nbuf]
        osem = bufsem[2 * nbuf:]
        base = _sc_wid() * per_w
        pltpu.sync_copy(idx_hbm.at[pl.ds(base, per_w)], idx_v)

        def gather(t, b):
            return pltpu.make_async_copy(
                tab_hbm.at[idx_v.at[pl.ds(t * KV_G, KV_G)]], bufs[b], gsem[b])

        def out_cp(t, b):
            return pltpu.make_async_copy(
                bufs[b], out_hbm.at[pl.ds(base + t * KV_G, KV_G)], osem[b])

        for b in range(nbuf):
            gather(b, b).start()

        def step(g, _):
            for b in range(nbuf):
                t = g * nbuf + b
                gather(t, b).wait()
                out_cp(t, b).start()
                nxt = t + nbuf

                @pl.when(nxt < n_g)
                def _(b=b, t=t, nxt=nxt):
                    out_cp(t, b).wait()
                    gather(nxt, b).start()
            return 0

        lax.fori_loop(0, n_g // nbuf, step, 0)
        for b in range(nbuf):
            out_cp(n_g - nbuf + b, b).wait()

    return body(kvtab, gidx)


def _pos_gather(pbh_t, pbase):
    blks = PER_W // BZ

    @functools.partial(
        pl.kernel,
        out_type=jax.ShapeDtypeStruct((H * NBLK, WIN, BZ), jnp.float32),
        mesh=_sc_mesh(),
        scratch_types=[
            pltpu.VMEM((H * PB_ROWS,), jnp.float32),
            pltpu.VMEM((PER_W,), jnp.int32),
            pltpu.VMEM((blks, WIN, BZ), jnp.float32),
        ],
        compiler_params=pltpu.CompilerParams(needs_layout_passes=False),
    )
    def body(tab_hbm, idx_hbm, out_hbm, tab_v, idx_v, out_v):
        wid = _sc_wid()
        pltpu.sync_copy(tab_hbm, tab_v)
        pltpu.sync_copy(idx_hbm.at[pl.ds(wid * PER_W, PER_W)], idx_v)

        for blk in range(blks):
            def step(t, _, blk=blk):
                iv = idx_v[pl.ds((blk * (BZ // 16) + t) * 16, 16)]
                for j in range(WIN):
                    out_v[blk, j, pl.ds(t * 16, 16)] = \
                        plsc.load_gather(tab_v, [iv + j])
                return 0

            lax.fori_loop(0, BZ // 16, step, 0)
        pltpu.sync_copy(out_v, out_hbm.at[pl.ds(wid * blks, blks)])

    return body(pbh_t, pbase)


def kernel(x, qkv_w, qkv_b, out_w, out_b, pb_w0, pb_b0, pb_w1, pb_b1, pb_w2, pb_b2):
    cst = _get_consts()
    gidx = jnp.asarray(cst["gidx"])
    pbase = jnp.asarray(cst["pbase"])
    mask = jnp.asarray(cst["mask"])
    i4 = jnp.asarray(cst["i4"])

    w4 = qkv_w.reshape(H, D, 3, C)
    qw = w4[:, :, 0, :].reshape(H * D, C)
    kvw = jnp.transpose(w4[:, :, 1:, :], (0, 2, 1, 3)).reshape(2 * H * D, C)
    w_r = jnp.concatenate([qw, kvw], axis=0)
    b4 = qkv_b.reshape(H, D, 3)
    qb = b4[:, :, 0].reshape(-1)
    kvb = jnp.transpose(b4[:, :, 1:], (0, 2, 1)).reshape(-1)
    b_r = jnp.concatenate([qb, kvb])[None, :]

    q3, kv2d = _qkv_proj(x[0], w_r, b_r)
    kvtab = kv2d.reshape(N * H, 2 * D)

    pbh = _pos_mlp(pb_w0.T, pb_b0[None, :], pb_w1, pb_b1[None, :],
                   pb_w2, pb_b2[None, :])
    pbh_t = pbh.T.reshape(-1)

    hg = H // 2
    nsh = NSEL // 2
    kvsel_a = _kv_gather(kvtab, gidx[:nsh], nsh)
    kvsel_b = _kv_gather(kvtab, gidx[nsh:], nsh)
    psel4 = _pos_gather(pbh_t, pbase)

    a3a = _attention(q3, kvsel_a, psel4, mask, i4, 0, hg)
    a3b = _attention(q3, kvsel_b, psel4, mask, i4, hg, hg)
    y = _out_proj(a3a, a3b, out_w, out_b[None, :])
    return y[None]
